# Initial kernel scaffold; baseline (speedup 1.0000x reference)
import functools

import jax
import jax.numpy as jnp
from jax import lax
from jax.experimental import pallas as pl
from jax.experimental.pallas import tpu as pltpu
from jax.experimental.pallas import tpu_sc as plsc

BATCH = 16384
SEQ = 200
EMB = 64

NC = 2
NS = 16
NW = NC * NS

G = 128
K = 8
CHUNK = G * K

TOTAL = BATCH * SEQ
GROUPS = TOTAL // G
GROUPS_PER_W = GROUPS // NW
ITERS = GROUPS_PER_W // K


def _body(idx_hbm, table_hbm, out_hbm, idx_v, rows_v, sem):
    wid = lax.axis_index("s") * NC + lax.axis_index("c")
    g_base = wid * GROUPS_PER_W

    def step(it, _):
        g0 = g_base + it * K
        pltpu.sync_copy(idx_hbm.at[pl.ds(g0, K)], idx_v)
        handles = []
        for j in range(K):
            handles.append(
                pltpu.async_copy(
                    table_hbm.at[idx_v.at[j]],
                    rows_v.at[pl.ds(j * G, G)],
                    sem,
                )
            )
        for h in handles:
            h.wait()
        pltpu.sync_copy(rows_v, out_hbm.at[pl.ds(g0 * G, CHUNK)])
        return 0

    lax.fori_loop(0, ITERS, step, 0)


def kernel(visit_segments, table):
    idx = visit_segments.reshape(GROUPS, G).astype(jnp.int32)
    mesh = plsc.VectorSubcoreMesh(
        core_axis_name="c", subcore_axis_name="s",
        num_cores=NC, num_subcores=NS,
    )
    grab = pl.kernel(
        _body,
        out_type=jax.ShapeDtypeStruct((TOTAL, EMB), jnp.float32),
        mesh=mesh,
        scratch_types=[
            pltpu.VMEM((K, G), jnp.int32),
            pltpu.VMEM((CHUNK, EMB), jnp.float32),
            pltpu.SemaphoreType.DMA,
        ],
    )
    out = grab(idx, table)
    return out.reshape(BATCH, SEQ, EMB)


# speedup vs baseline: 4.1483x; 4.1483x over previous
import functools

import jax
import jax.numpy as jnp
from jax import lax
from jax.experimental import pallas as pl
from jax.experimental.pallas import tpu as pltpu
from jax.experimental.pallas import tpu_sc as plsc

BATCH = 16384
SEQ = 200
EMB = 64

NC = 2
NS = 16
NW = NC * NS

G = 128
K = 8
CHUNK = G * K

TOTAL = BATCH * SEQ
GROUPS = TOTAL // G
GROUPS_PER_W = GROUPS // NW
ITERS = GROUPS_PER_W // K


def _body(idx_hbm, table_hbm, out_hbm, idx_v, rows_v, sem):
    wid = lax.axis_index("s") * NC + lax.axis_index("c")
    g_base = wid * GROUPS_PER_W

    def step(it, _):
        g0 = g_base + it * K
        pltpu.sync_copy(idx_hbm.at[pl.ds(g0, K)], idx_v)
        handles = []
        for j in range(K):
            handles.append(
                pltpu.async_copy(
                    table_hbm.at[idx_v.at[j]],
                    rows_v.at[pl.ds(j * G, G)],
                    sem,
                )
            )
        for h in handles:
            h.wait()
        pltpu.sync_copy(rows_v, out_hbm.at[pl.ds(g0 * G, CHUNK)])
        return 0

    lax.fori_loop(0, ITERS, step, 0)


def kernel(visit_segments, table):
    idx = visit_segments.reshape(GROUPS, G).astype(jnp.int32)
    mesh = plsc.VectorSubcoreMesh(
        core_axis_name="c", subcore_axis_name="s",
        num_cores=NC, num_subcores=NS,
    )
    grab = pl.kernel(
        _body,
        out_type=jax.ShapeDtypeStruct((TOTAL, EMB), jnp.float32),
        mesh=mesh,
        scratch_types=[
            pltpu.VMEM((K, G), jnp.int32),
            pltpu.VMEM((CHUNK, EMB), jnp.float32),
            pltpu.SemaphoreType.DMA,
        ],
        compiler_params=pltpu.CompilerParams(use_tc_tiling_on_sc=False),
    )
    out = grab(idx, table)
    return out.reshape(BATCH, SEQ, EMB)
